# baseline (speedup 1.0000x reference)
import functools

import jax
import jax.numpy as jnp
from jax import lax
from jax.experimental import pallas as pl
from jax.experimental.pallas import tpu as pltpu
from jax.experimental.pallas import tpu_sc as plsc

MEM = 100000
DIM = 128
BS = 1024
K = 512
TEMP = 0.07

ROWS_PER_TILE = 2048
HALF = ROWS_PER_TILE // 2
NTILES = 49
NWORDS = NTILES * HALF * BS

NWORKERS = 32
BPW = BS // NWORKERS
NEG_PER_W = BPW * K
NCHUNK = NEG_PER_W // 128
WINDOW = 8


def _neg_sample_draw():
    return jax.random.randint(
        jax.random.key(1), (BS, K), 0, MEM - 1).astype(jnp.int32)


def _pack_bf16_pair(se, so):
    be = lax.bitcast_convert_type(se, jnp.int32)
    bo = lax.bitcast_convert_type(so, jnp.int32)
    re = lax.shift_right_logical(be + 0x8000, 16)
    ro = lax.shift_right_logical(bo + 0x8000, 16)
    return (ro << 16) | re


def _tc_body(sy_ref, sperm_ref, starts_ref,
             vemb_ref, aemb_ref, m1_ref, m2_ref,
             s1_ref, s2_ref, n1_ref, n2_ref,
             vn_ref, an_ref, vsc_ref, asc_ref):
    t = pl.program_id(0)

    @pl.when(t == 0)
    def _():
        v = vemb_ref[...]
        vn = v / jnp.maximum(jnp.sqrt(jnp.sum(v * v, axis=1, keepdims=True)), 1e-12)
        vn_ref[...] = vn
        vsc_ref[...] = (vn * (1.0 / TEMP)).astype(jnp.bfloat16)
        a = aemb_ref[...]
        an = a / jnp.maximum(jnp.sqrt(jnp.sum(a * a, axis=1, keepdims=True)), 1e-12)
        an_ref[...] = an
        asc_ref[...] = (an * (1.0 / TEMP)).astype(jnp.bfloat16)

    m1 = m1_ref[...]
    m2 = m2_ref[...]
    dn = (((1,), (1,)), ((), ()))
    for cb in range(HALF // 128):
        slo = slice(cb * 128, (cb + 1) * 128)
        shi = slice(HALF + cb * 128, HALF + (cb + 1) * 128)
        s1_ref[pl.ds(cb * BS, BS), :] = _pack_bf16_pair(
            lax.dot_general(asc_ref[...], m1[slo].astype(jnp.bfloat16),
                            dn, preferred_element_type=jnp.float32),
            lax.dot_general(asc_ref[...], m1[shi].astype(jnp.bfloat16),
                            dn, preferred_element_type=jnp.float32))
        s2_ref[pl.ds(cb * BS, BS), :] = _pack_bf16_pair(
            lax.dot_general(vsc_ref[...], m2[slo].astype(jnp.bfloat16),
                            dn, preferred_element_type=jnp.float32),
            lax.dot_general(vsc_ref[...], m2[shi].astype(jnp.bfloat16),
                            dn, preferred_element_type=jnp.float32))
    n1_ref[...] = m1
    n2_ref[...] = m2

    lo = starts_ref[t]
    hi = starts_ref[t + 1]
    base = t * ROWS_PER_TILE

    def upd_at(j):
        row = sy_ref[j] - base
        b = sperm_ref[j]

        def bank(m_ref, n_ref, e_ref):
            l = m_ref[pl.ds(row, 1), :] * 0.5 + e_ref[pl.ds(b, 1), :] * 0.5
            l = l / jnp.maximum(jnp.sqrt(jnp.sum(l * l)), 1e-12)
            n_ref[pl.ds(row, 1), :] = l

        bank(m1_ref, n1_ref, vn_ref)
        bank(m2_ref, n2_ref, an_ref)

    CAP = 64

    def upd_cap(i, carry):
        jj = lo + i

        @pl.when(jj < hi)
        def _():
            upd_at(jj)
        return carry

    lax.fori_loop(0, CAP, upd_cap, 0, unroll=4)

    def upd_rest(j, carry):
        upd_at(j)
        return carry

    lax.fori_loop(lo + CAP, hi, upd_rest, 0)


_TC_GRID = pltpu.PrefetchScalarGridSpec(
    num_scalar_prefetch=3,
    grid=(NTILES,),
    in_specs=[
        pl.BlockSpec((BS, DIM), lambda t, *_: (0, 0)),
        pl.BlockSpec((BS, DIM), lambda t, *_: (0, 0)),
        pl.BlockSpec((ROWS_PER_TILE, DIM), lambda t, *_: (t, 0)),
        pl.BlockSpec((ROWS_PER_TILE, DIM), lambda t, *_: (t, 0)),
    ],
    out_specs=[
        pl.BlockSpec((HALF // 128 * BS, DIM), lambda t, *_: (t, 0)),
        pl.BlockSpec((HALF // 128 * BS, DIM), lambda t, *_: (t, 0)),
        pl.BlockSpec((ROWS_PER_TILE, DIM), lambda t, *_: (t, 0)),
        pl.BlockSpec((ROWS_PER_TILE, DIM), lambda t, *_: (t, 0)),
    ],
    scratch_shapes=[
        pltpu.VMEM((BS, DIM), jnp.float32),
        pltpu.VMEM((BS, DIM), jnp.float32),
        pltpu.VMEM((BS, DIM), jnp.bfloat16),
        pltpu.VMEM((BS, DIM), jnp.bfloat16),
    ],
)


def _sc_body(s2_ref, s1_ref, y_ref, r_ref,
             outn_ref, outp_ref,
             rb_v, y_v, fidx_v, half_v, fidxp_v, halfp_v,
             gath_v, gathf_v, gathp_v, gathpf_v, sem):
    cid = lax.axis_index("c")
    sid = lax.axis_index("s")
    wid = sid * 2 + cid
    base_b = wid * BPW

    pltpu.sync_copy(r_ref.at[pl.ds(base_b, BPW), :], rb_v)
    pltpu.sync_copy(y_ref.at[pl.ds(base_b, BPW)], y_v)

    ones = jnp.ones((16,), jnp.int32)
    c3 = jnp.full((16,), 3, jnp.int32)
    c7 = jnp.full((16,), 7, jnp.int32)
    c10 = jnp.full((16,), 10, jnp.int32)
    c11 = jnp.full((16,), 11, jnp.int32)
    c16 = jnp.full((16,), 16, jnp.int32)
    c17 = jnp.full((16,), 17, jnp.int32)
    c127 = jnp.full((16,), 127, jnp.int32)
    c1023 = jnp.full((16,), 1023, jnp.int32)
    cmask = jnp.full((16,), 0xFFFF, jnp.int32)

    def word_of(jv, b7):
        q = jv & c1023
        g = (lax.shift_left(lax.shift_right_logical(jv, c11), c3)
             + lax.shift_right_logical(q, c7))
        return lax.shift_left(g, c17) + (q & c127) + b7

    def half_of(jv):
        return lax.shift_right_logical(jv, c10) & ones

    for h in range(BPW // 16):
        yv = y_v[pl.ds(h * 16, 16)]
        b7 = lax.shift_left(
            jnp.full((16,), base_b + h * 16, jnp.int32)
            + lax.iota(jnp.int32, 16), c7)
        fidxp_v[pl.ds(h * 16, 16)] = word_of(yv, b7)
        halfp_v[pl.ds(h * 16, 16)] = half_of(yv)

        for j in range(16):
            bl = h * 16 + j
            tbvec = jnp.full((16,), yv[j], jnp.int32)
            bvec7 = jnp.full((16,), (base_b + bl) << 7, jnp.int32)

            def fill_k(kc, c2, bl=bl, tbvec=tbvec, bvec7=bvec7):
                rv = rb_v[bl, pl.ds(kc * 16, 16)]
                jv = jnp.where(rv >= tbvec, rv + ones, rv)
                off = bl * K + kc * 16
                fidx_v[pl.ds(off, 16)] = word_of(jv, bvec7)
                half_v[pl.ds(off, 16)] = half_of(jv)
                return c2

            lax.fori_loop(0, K // 16, fill_k, 0)

    for m, sref in ((0, s2_ref), (1, s1_ref)):
        def gbody(c, carry, sref=sref):
            pltpu.async_copy(
                sref.at[fidx_v.at[pl.ds(c * 128, 128)]],
                gath_v.at[pl.ds(c * 128, 128)], sem)

            @pl.when(c >= WINDOW)
            def _():
                cw = c - WINDOW
                pltpu.make_async_copy(
                    sref.at[fidx_v.at[pl.ds(cw * 128, 128)]],
                    gath_v.at[pl.ds(cw * 128, 128)], sem).wait()
            return carry

        lax.fori_loop(0, NCHUNK, gbody, 0)
        for w in range(WINDOW):
            cw = NCHUNK - WINDOW + w
            pltpu.make_async_copy(
                sref.at[fidx_v.at[pl.ds(cw * 128, 128)]],
                gath_v.at[pl.ds(cw * 128, 128)], sem).wait()
        pltpu.async_copy(sref.at[fidxp_v], gathp_v, sem).wait()

        def unpack(c, carry):
            g = gath_v[pl.ds(c * 16, 16)]
            sh = half_v[pl.ds(c * 16, 16)] << jnp.full((16,), 4, jnp.int32)
            bits = (lax.shift_right_logical(g, sh) & cmask) << c16
            gathf_v[pl.ds(c * 16, 16)] = lax.bitcast_convert_type(bits, jnp.float32)
            return carry

        lax.fori_loop(0, NEG_PER_W // 16, unpack, 0)
        for h in range(BPW // 16):
            g = gathp_v[pl.ds(h * 16, 16)]
            sh = halfp_v[pl.ds(h * 16, 16)] << jnp.full((16,), 4, jnp.int32)
            bits = (lax.shift_right_logical(g, sh) & cmask) << c16
            gathpf_v[pl.ds(h * 16, 16)] = lax.bitcast_convert_type(bits, jnp.float32)

        pltpu.sync_copy(gathf_v, outn_ref.at[m, pl.ds(wid * NEG_PER_W, NEG_PER_W)])
        pltpu.sync_copy(gathpf_v, outp_ref.at[m, pl.ds(wid * BPW, BPW)])


_sc_extract_cached = None


def _sc_extract(*args):
    global _sc_extract_cached
    if _sc_extract_cached is None:
        _sc_extract_cached = functools.partial(
            pl.kernel,
            out_type=(
                jax.ShapeDtypeStruct((2, BS * K), jnp.float32),
                jax.ShapeDtypeStruct((2, BS), jnp.float32),
            ),
            mesh=plsc.VectorSubcoreMesh(core_axis_name="c", subcore_axis_name="s"),
            scratch_types=[
                pltpu.VMEM((BPW, K), jnp.int32),
                pltpu.VMEM((BPW,), jnp.int32),
                pltpu.VMEM((NEG_PER_W,), jnp.int32),
                pltpu.VMEM((NEG_PER_W,), jnp.int32),
                pltpu.VMEM((BPW,), jnp.int32),
                pltpu.VMEM((BPW,), jnp.int32),
                pltpu.VMEM((NEG_PER_W,), jnp.int32),
                pltpu.VMEM((NEG_PER_W,), jnp.float32),
                pltpu.VMEM((BPW,), jnp.int32),
                pltpu.VMEM((BPW,), jnp.float32),
                pltpu.SemaphoreType.DMA,
            ],
        )(_sc_body)
    return _sc_extract_cached(*args)


def kernel(video_emb, audio_emb, y, view1_mem, view2_mem):
    y = y.astype(jnp.int32)

    bidx = jnp.arange(BS, dtype=jnp.int32)
    skey = y * BS + bidx
    rank = jnp.sum((skey[None, :] < skey[:, None]).astype(jnp.int32), axis=1)
    onehot = (rank[:, None] == bidx[None, :]).astype(jnp.int32)
    sy = jnp.sum(onehot * y[:, None], axis=0)
    perm = jnp.sum(onehot * bidx[:, None], axis=0)
    tiles = jnp.arange(NTILES + 1, dtype=jnp.int32)
    starts = jnp.sum(
        (y[None, :] < (tiles * ROWS_PER_TILE)[:, None]).astype(jnp.int32),
        axis=1)

    s1, s2, n1, n2 = pl.pallas_call(
        _tc_body,
        grid_spec=_TC_GRID,
        out_shape=[
            jax.ShapeDtypeStruct((NWORDS // 128, DIM), jnp.int32),
            jax.ShapeDtypeStruct((NWORDS // 128, DIM), jnp.int32),
            jax.ShapeDtypeStruct((MEM, DIM), jnp.float32),
            jax.ShapeDtypeStruct((MEM, DIM), jnp.float32),
        ],
    )(sy, perm, starts, video_emb, audio_emb, view1_mem, view2_mem)

    negs, pos = _sc_extract(
        s2.reshape(-1), s1.reshape(-1), y, _neg_sample_draw())

    scores = jnp.concatenate(
        [pos.reshape(2, BS, 1), negs.reshape(2, BS, K)], axis=2)
    return scores, n1, n2


# speedup vs baseline: 1.3479x; 1.3479x over previous
import functools

import jax
import jax.numpy as jnp
from jax import lax
from jax.experimental import pallas as pl
from jax.experimental.pallas import tpu as pltpu
from jax.experimental.pallas import tpu_sc as plsc

MEM = 100000
DIM = 128
BS = 1024
K = 512
TEMP = 0.07

ROWS_PER_TILE = 2048
HALF = ROWS_PER_TILE // 2
NTILES = 49
NWORDS = NTILES * HALF * BS

NWORKERS = 32
BPW = BS // NWORKERS
NEG_PER_W = BPW * K
NCHUNK = NEG_PER_W // 128
WINDOW = 8


def _neg_sample_draw():
    return jax.random.randint(
        jax.random.key(1), (BS, K), 0, MEM - 1).astype(jnp.int32)


def _pack_bf16_pair(se, so):
    be = lax.bitcast_convert_type(se, jnp.int32)
    bo = lax.bitcast_convert_type(so, jnp.int32)
    re = lax.shift_right_logical(be + 0x8000, 16)
    ro = lax.shift_right_logical(bo + 0x8000, 16)
    return (ro << 16) | re


def _tc_body(sy_ref, sperm_ref, starts_ref,
             vemb_ref, aemb_ref, m1_ref, m2_ref,
             s1_ref, s2_ref, n1_ref, n2_ref,
             vn_ref, an_ref, vsc_ref, asc_ref):
    t = pl.program_id(0)

    @pl.when(t == 0)
    def _():
        v = vemb_ref[...]
        vn = v / jnp.maximum(jnp.sqrt(jnp.sum(v * v, axis=1, keepdims=True)), 1e-12)
        vn_ref[...] = vn
        vsc_ref[...] = (vn * (1.0 / TEMP)).astype(jnp.bfloat16)
        a = aemb_ref[...]
        an = a / jnp.maximum(jnp.sqrt(jnp.sum(a * a, axis=1, keepdims=True)), 1e-12)
        an_ref[...] = an
        asc_ref[...] = (an * (1.0 / TEMP)).astype(jnp.bfloat16)

    m1 = m1_ref[...]
    m2 = m2_ref[...]
    dn = (((1,), (1,)), ((), ()))
    for cb in range(HALF // 128):
        slo = slice(cb * 128, (cb + 1) * 128)
        shi = slice(HALF + cb * 128, HALF + (cb + 1) * 128)
        s1_ref[pl.ds(cb * BS, BS), :] = _pack_bf16_pair(
            lax.dot_general(asc_ref[...], m1[slo].astype(jnp.bfloat16),
                            dn, preferred_element_type=jnp.float32),
            lax.dot_general(asc_ref[...], m1[shi].astype(jnp.bfloat16),
                            dn, preferred_element_type=jnp.float32))
        s2_ref[pl.ds(cb * BS, BS), :] = _pack_bf16_pair(
            lax.dot_general(vsc_ref[...], m2[slo].astype(jnp.bfloat16),
                            dn, preferred_element_type=jnp.float32),
            lax.dot_general(vsc_ref[...], m2[shi].astype(jnp.bfloat16),
                            dn, preferred_element_type=jnp.float32))
    n1_ref[...] = m1
    n2_ref[...] = m2

    lo = starts_ref[t]
    hi = starts_ref[t + 1]
    base = t * ROWS_PER_TILE

    def upd(j, carry):
        row = sy_ref[j] - base
        b = sperm_ref[j]
        old = jnp.concatenate(
            [m1_ref[pl.ds(row, 1), :], m2_ref[pl.ds(row, 1), :]], axis=0)
        emb = jnp.concatenate(
            [vn_ref[pl.ds(b, 1), :], an_ref[pl.ds(b, 1), :]], axis=0)
        l = old * 0.5 + emb * 0.5
        s = jnp.sum(l * l, axis=1, keepdims=True)
        l = l / jnp.maximum(jnp.sqrt(s), 1e-12)
        n1_ref[pl.ds(row, 1), :] = l[0:1, :]
        n2_ref[pl.ds(row, 1), :] = l[1:2, :]
        return carry

    lax.fori_loop(lo, hi, upd, 0)


_TC_GRID = pltpu.PrefetchScalarGridSpec(
    num_scalar_prefetch=3,
    grid=(NTILES,),
    in_specs=[
        pl.BlockSpec((BS, DIM), lambda t, *_: (0, 0)),
        pl.BlockSpec((BS, DIM), lambda t, *_: (0, 0)),
        pl.BlockSpec((ROWS_PER_TILE, DIM), lambda t, *_: (t, 0)),
        pl.BlockSpec((ROWS_PER_TILE, DIM), lambda t, *_: (t, 0)),
    ],
    out_specs=[
        pl.BlockSpec((HALF // 128 * BS, DIM), lambda t, *_: (t, 0)),
        pl.BlockSpec((HALF // 128 * BS, DIM), lambda t, *_: (t, 0)),
        pl.BlockSpec((ROWS_PER_TILE, DIM), lambda t, *_: (t, 0)),
        pl.BlockSpec((ROWS_PER_TILE, DIM), lambda t, *_: (t, 0)),
    ],
    scratch_shapes=[
        pltpu.VMEM((BS, DIM), jnp.float32),
        pltpu.VMEM((BS, DIM), jnp.float32),
        pltpu.VMEM((BS, DIM), jnp.bfloat16),
        pltpu.VMEM((BS, DIM), jnp.bfloat16),
    ],
)


def _sc_body(s2_ref, s1_ref, y_ref, r_ref,
             outn_ref, outp_ref,
             rb_v, y_v, fidx_v, half_v, fidxp_v, halfp_v,
             gath_v, gathf_v, gathp_v, gathpf_v, sem):
    cid = lax.axis_index("c")
    sid = lax.axis_index("s")
    wid = sid * 2 + cid
    base_b = wid * BPW

    pltpu.sync_copy(r_ref.at[pl.ds(base_b, BPW), :], rb_v)
    pltpu.sync_copy(y_ref.at[pl.ds(base_b, BPW)], y_v)

    ones = jnp.ones((16,), jnp.int32)
    c3 = jnp.full((16,), 3, jnp.int32)
    c7 = jnp.full((16,), 7, jnp.int32)
    c10 = jnp.full((16,), 10, jnp.int32)
    c11 = jnp.full((16,), 11, jnp.int32)
    c16 = jnp.full((16,), 16, jnp.int32)
    c17 = jnp.full((16,), 17, jnp.int32)
    c127 = jnp.full((16,), 127, jnp.int32)
    c1023 = jnp.full((16,), 1023, jnp.int32)
    cmask = jnp.full((16,), 0xFFFF, jnp.int32)

    def word_of(jv, b7):
        q = jv & c1023
        g = (lax.shift_left(lax.shift_right_logical(jv, c11), c3)
             + lax.shift_right_logical(q, c7))
        return lax.shift_left(g, c17) + (q & c127) + b7

    def half_of(jv):
        return lax.shift_right_logical(jv, c10) & ones

    for h in range(BPW // 16):
        yv = y_v[pl.ds(h * 16, 16)]
        b7 = lax.shift_left(
            jnp.full((16,), base_b + h * 16, jnp.int32)
            + lax.iota(jnp.int32, 16), c7)
        fidxp_v[pl.ds(h * 16, 16)] = word_of(yv, b7)
        halfp_v[pl.ds(h * 16, 16)] = half_of(yv)

        for j in range(16):
            bl = h * 16 + j
            tbvec = jnp.full((16,), yv[j], jnp.int32)
            bvec7 = jnp.full((16,), (base_b + bl) << 7, jnp.int32)

            def fill_k(kc, c2, bl=bl, tbvec=tbvec, bvec7=bvec7):
                rv = rb_v[bl, pl.ds(kc * 16, 16)]
                jv = jnp.where(rv >= tbvec, rv + ones, rv)
                off = bl * K + kc * 16
                fidx_v[pl.ds(off, 16)] = word_of(jv, bvec7)
                half_v[pl.ds(off, 16)] = half_of(jv)
                return c2

            lax.fori_loop(0, K // 16, fill_k, 0)

    for m, sref in ((0, s2_ref), (1, s1_ref)):
        def gbody(c, carry, sref=sref):
            pltpu.async_copy(
                sref.at[fidx_v.at[pl.ds(c * 128, 128)]],
                gath_v.at[pl.ds(c * 128, 128)], sem)

            @pl.when(c >= WINDOW)
            def _():
                cw = c - WINDOW
                pltpu.make_async_copy(
                    sref.at[fidx_v.at[pl.ds(cw * 128, 128)]],
                    gath_v.at[pl.ds(cw * 128, 128)], sem).wait()
            return carry

        lax.fori_loop(0, NCHUNK, gbody, 0)
        for w in range(WINDOW):
            cw = NCHUNK - WINDOW + w
            pltpu.make_async_copy(
                sref.at[fidx_v.at[pl.ds(cw * 128, 128)]],
                gath_v.at[pl.ds(cw * 128, 128)], sem).wait()
        pltpu.async_copy(sref.at[fidxp_v], gathp_v, sem).wait()

        def unpack(c, carry):
            g = gath_v[pl.ds(c * 16, 16)]
            sh = half_v[pl.ds(c * 16, 16)] << jnp.full((16,), 4, jnp.int32)
            bits = (lax.shift_right_logical(g, sh) & cmask) << c16
            gathf_v[pl.ds(c * 16, 16)] = lax.bitcast_convert_type(bits, jnp.float32)
            return carry

        lax.fori_loop(0, NEG_PER_W // 16, unpack, 0)
        for h in range(BPW // 16):
            g = gathp_v[pl.ds(h * 16, 16)]
            sh = halfp_v[pl.ds(h * 16, 16)] << jnp.full((16,), 4, jnp.int32)
            bits = (lax.shift_right_logical(g, sh) & cmask) << c16
            gathpf_v[pl.ds(h * 16, 16)] = lax.bitcast_convert_type(bits, jnp.float32)

        pltpu.sync_copy(gathf_v, outn_ref.at[m, pl.ds(wid * NEG_PER_W, NEG_PER_W)])
        pltpu.sync_copy(gathpf_v, outp_ref.at[m, pl.ds(wid * BPW, BPW)])


_sc_extract_cached = None


def _sc_extract(*args):
    global _sc_extract_cached
    if _sc_extract_cached is None:
        _sc_extract_cached = functools.partial(
            pl.kernel,
            out_type=(
                jax.ShapeDtypeStruct((2, BS * K), jnp.float32),
                jax.ShapeDtypeStruct((2, BS), jnp.float32),
            ),
            mesh=plsc.VectorSubcoreMesh(core_axis_name="c", subcore_axis_name="s"),
            scratch_types=[
                pltpu.VMEM((BPW, K), jnp.int32),
                pltpu.VMEM((BPW,), jnp.int32),
                pltpu.VMEM((NEG_PER_W,), jnp.int32),
                pltpu.VMEM((NEG_PER_W,), jnp.int32),
                pltpu.VMEM((BPW,), jnp.int32),
                pltpu.VMEM((BPW,), jnp.int32),
                pltpu.VMEM((NEG_PER_W,), jnp.int32),
                pltpu.VMEM((NEG_PER_W,), jnp.float32),
                pltpu.VMEM((BPW,), jnp.int32),
                pltpu.VMEM((BPW,), jnp.float32),
                pltpu.SemaphoreType.DMA,
            ],
        )(_sc_body)
    return _sc_extract_cached(*args)


def kernel(video_emb, audio_emb, y, view1_mem, view2_mem):
    y = y.astype(jnp.int32)

    bidx = jnp.arange(BS, dtype=jnp.int32)
    skey = y * BS + bidx
    rank = jnp.sum((skey[None, :] < skey[:, None]).astype(jnp.int32), axis=1)
    onehot = (rank[:, None] == bidx[None, :]).astype(jnp.int32)
    sy = jnp.sum(onehot * y[:, None], axis=0)
    perm = jnp.sum(onehot * bidx[:, None], axis=0)
    tiles = jnp.arange(NTILES + 1, dtype=jnp.int32)
    starts = jnp.sum(
        (y[None, :] < (tiles * ROWS_PER_TILE)[:, None]).astype(jnp.int32),
        axis=1)

    s1, s2, n1, n2 = pl.pallas_call(
        _tc_body,
        grid_spec=_TC_GRID,
        out_shape=[
            jax.ShapeDtypeStruct((NWORDS // 128, DIM), jnp.int32),
            jax.ShapeDtypeStruct((NWORDS // 128, DIM), jnp.int32),
            jax.ShapeDtypeStruct((MEM, DIM), jnp.float32),
            jax.ShapeDtypeStruct((MEM, DIM), jnp.float32),
        ],
    )(sy, perm, starts, video_emb, audio_emb, view1_mem, view2_mem)

    negs, pos = _sc_extract(
        s2.reshape(-1), s1.reshape(-1), y, _neg_sample_draw())

    scores = jnp.concatenate(
        [pos.reshape(2, BS, 1), negs.reshape(2, BS, K)], axis=2)
    return scores, n1, n2


# speedup vs baseline: 1.3780x; 1.0223x over previous
import functools

import jax
import jax.numpy as jnp
from jax import lax
from jax.experimental import pallas as pl
from jax.experimental.pallas import tpu as pltpu
from jax.experimental.pallas import tpu_sc as plsc

MEM = 100000
DIM = 128
BS = 1024
K = 512
TEMP = 0.07

ROWS_PER_TILE = 2048
HALF = ROWS_PER_TILE // 2
NTILES = 49
NWORDS = NTILES * HALF * BS

NWORKERS = 32
BPW = BS // NWORKERS
NEG_PER_W = BPW * K
NCHUNK = NEG_PER_W // 128
WINDOW = 16


def _neg_sample_draw():
    return jax.random.randint(
        jax.random.key(1), (BS, K), 0, MEM - 1).astype(jnp.int32)


def _pack_bf16_pair(se, so):
    be = lax.bitcast_convert_type(se, jnp.int32)
    bo = lax.bitcast_convert_type(so, jnp.int32)
    re = lax.shift_right_logical(be + 0x8000, 16)
    ro = lax.shift_right_logical(bo + 0x8000, 16)
    return (ro << 16) | re


def _tc_body(sy_ref, sperm_ref, starts_ref,
             vemb_ref, aemb_ref, m1_ref, m2_ref,
             s1_ref, s2_ref, n1_ref, n2_ref,
             vn_ref, an_ref, vsc_ref, asc_ref):
    t = pl.program_id(0)

    @pl.when(t == 0)
    def _():
        v = vemb_ref[...]
        vn = v / jnp.maximum(jnp.sqrt(jnp.sum(v * v, axis=1, keepdims=True)), 1e-12)
        vn_ref[...] = vn
        vsc_ref[...] = (vn * (1.0 / TEMP)).astype(jnp.bfloat16)
        a = aemb_ref[...]
        an = a / jnp.maximum(jnp.sqrt(jnp.sum(a * a, axis=1, keepdims=True)), 1e-12)
        an_ref[...] = an
        asc_ref[...] = (an * (1.0 / TEMP)).astype(jnp.bfloat16)

    m1 = m1_ref[...]
    m2 = m2_ref[...]
    dn = (((1,), (1,)), ((), ()))
    for cb in range(HALF // 128):
        slo = slice(cb * 128, (cb + 1) * 128)
        shi = slice(HALF + cb * 128, HALF + (cb + 1) * 128)
        s1_ref[pl.ds(cb * BS, BS), :] = _pack_bf16_pair(
            lax.dot_general(asc_ref[...], m1[slo].astype(jnp.bfloat16),
                            dn, preferred_element_type=jnp.float32),
            lax.dot_general(asc_ref[...], m1[shi].astype(jnp.bfloat16),
                            dn, preferred_element_type=jnp.float32))
        s2_ref[pl.ds(cb * BS, BS), :] = _pack_bf16_pair(
            lax.dot_general(vsc_ref[...], m2[slo].astype(jnp.bfloat16),
                            dn, preferred_element_type=jnp.float32),
            lax.dot_general(vsc_ref[...], m2[shi].astype(jnp.bfloat16),
                            dn, preferred_element_type=jnp.float32))
    n1_ref[...] = m1
    n2_ref[...] = m2

    lo = starts_ref[t]
    hi = starts_ref[t + 1]
    base = t * ROWS_PER_TILE

    def upd(j, carry):
        row = sy_ref[j] - base
        b = sperm_ref[j]
        old = jnp.concatenate(
            [m1_ref[pl.ds(row, 1), :], m2_ref[pl.ds(row, 1), :]], axis=0)
        emb = jnp.concatenate(
            [vn_ref[pl.ds(b, 1), :], an_ref[pl.ds(b, 1), :]], axis=0)
        l = old * 0.5 + emb * 0.5
        s = jnp.sum(l * l, axis=1, keepdims=True)
        l = l / jnp.maximum(jnp.sqrt(s), 1e-12)
        n1_ref[pl.ds(row, 1), :] = l[0:1, :]
        n2_ref[pl.ds(row, 1), :] = l[1:2, :]
        return carry

    lax.fori_loop(lo, hi, upd, 0)


_TC_GRID = pltpu.PrefetchScalarGridSpec(
    num_scalar_prefetch=3,
    grid=(NTILES,),
    in_specs=[
        pl.BlockSpec((BS, DIM), lambda t, *_: (0, 0)),
        pl.BlockSpec((BS, DIM), lambda t, *_: (0, 0)),
        pl.BlockSpec((ROWS_PER_TILE, DIM), lambda t, *_: (t, 0)),
        pl.BlockSpec((ROWS_PER_TILE, DIM), lambda t, *_: (t, 0)),
    ],
    out_specs=[
        pl.BlockSpec((HALF // 128 * BS, DIM), lambda t, *_: (t, 0)),
        pl.BlockSpec((HALF // 128 * BS, DIM), lambda t, *_: (t, 0)),
        pl.BlockSpec((ROWS_PER_TILE, DIM), lambda t, *_: (t, 0)),
        pl.BlockSpec((ROWS_PER_TILE, DIM), lambda t, *_: (t, 0)),
    ],
    scratch_shapes=[
        pltpu.VMEM((BS, DIM), jnp.float32),
        pltpu.VMEM((BS, DIM), jnp.float32),
        pltpu.VMEM((BS, DIM), jnp.bfloat16),
        pltpu.VMEM((BS, DIM), jnp.bfloat16),
    ],
)


def _sc_body(s2_ref, s1_ref, y_ref, r_ref,
             outn_ref, outp_ref,
             rb_v, y_v, fidx_v, half_v, fidxp_v, halfp_v,
             gath_v, gathf_v, gathp_v, gathpf_v, sem):
    cid = lax.axis_index("c")
    sid = lax.axis_index("s")
    wid = sid * 2 + cid
    base_b = wid * BPW

    pltpu.sync_copy(r_ref.at[pl.ds(base_b, BPW), :], rb_v)
    pltpu.sync_copy(y_ref.at[pl.ds(base_b, BPW)], y_v)

    ones = jnp.ones((16,), jnp.int32)
    c3 = jnp.full((16,), 3, jnp.int32)
    c7 = jnp.full((16,), 7, jnp.int32)
    c10 = jnp.full((16,), 10, jnp.int32)
    c11 = jnp.full((16,), 11, jnp.int32)
    c16 = jnp.full((16,), 16, jnp.int32)
    c17 = jnp.full((16,), 17, jnp.int32)
    c127 = jnp.full((16,), 127, jnp.int32)
    c1023 = jnp.full((16,), 1023, jnp.int32)
    cmask = jnp.full((16,), 0xFFFF, jnp.int32)

    def word_of(jv, b7):
        q = jv & c1023
        g = (lax.shift_left(lax.shift_right_logical(jv, c11), c3)
             + lax.shift_right_logical(q, c7))
        return lax.shift_left(g, c17) + (q & c127) + b7

    def half_of(jv):
        return lax.shift_right_logical(jv, c10) & ones

    for h in range(BPW // 16):
        yv = y_v[pl.ds(h * 16, 16)]
        b7 = lax.shift_left(
            jnp.full((16,), base_b + h * 16, jnp.int32)
            + lax.iota(jnp.int32, 16), c7)
        fidxp_v[pl.ds(h * 16, 16)] = word_of(yv, b7)
        halfp_v[pl.ds(h * 16, 16)] = half_of(yv)

        for j in range(16):
            bl = h * 16 + j
            tbvec = jnp.full((16,), yv[j], jnp.int32)
            bvec7 = jnp.full((16,), (base_b + bl) << 7, jnp.int32)

            def fill_k(kc, c2, bl=bl, tbvec=tbvec, bvec7=bvec7):
                rv = rb_v[bl, pl.ds(kc * 16, 16)]
                jv = jnp.where(rv >= tbvec, rv + ones, rv)
                off = bl * K + kc * 16
                fidx_v[pl.ds(off, 16)] = word_of(jv, bvec7)
                half_v[pl.ds(off, 16)] = half_of(jv)
                return c2

            lax.fori_loop(0, K // 16, fill_k, 0)

    for m, sref in ((0, s2_ref), (1, s1_ref)):
        def gbody(c, carry, sref=sref):
            pltpu.async_copy(
                sref.at[fidx_v.at[pl.ds(c * 128, 128)]],
                gath_v.at[pl.ds(c * 128, 128)], sem)

            @pl.when(c >= WINDOW)
            def _():
                cw = c - WINDOW
                pltpu.make_async_copy(
                    sref.at[fidx_v.at[pl.ds(cw * 128, 128)]],
                    gath_v.at[pl.ds(cw * 128, 128)], sem).wait()
            return carry

        lax.fori_loop(0, NCHUNK, gbody, 0)
        for w in range(WINDOW):
            cw = NCHUNK - WINDOW + w
            pltpu.make_async_copy(
                sref.at[fidx_v.at[pl.ds(cw * 128, 128)]],
                gath_v.at[pl.ds(cw * 128, 128)], sem).wait()
        pltpu.async_copy(sref.at[fidxp_v], gathp_v, sem).wait()

        def unpack(c, carry):
            g = gath_v[pl.ds(c * 16, 16)]
            sh = half_v[pl.ds(c * 16, 16)] << jnp.full((16,), 4, jnp.int32)
            bits = (lax.shift_right_logical(g, sh) & cmask) << c16
            gathf_v[pl.ds(c * 16, 16)] = lax.bitcast_convert_type(bits, jnp.float32)
            return carry

        lax.fori_loop(0, NEG_PER_W // 16, unpack, 0)
        for h in range(BPW // 16):
            g = gathp_v[pl.ds(h * 16, 16)]
            sh = halfp_v[pl.ds(h * 16, 16)] << jnp.full((16,), 4, jnp.int32)
            bits = (lax.shift_right_logical(g, sh) & cmask) << c16
            gathpf_v[pl.ds(h * 16, 16)] = lax.bitcast_convert_type(bits, jnp.float32)

        pltpu.sync_copy(gathf_v, outn_ref.at[m, pl.ds(wid * NEG_PER_W, NEG_PER_W)])
        pltpu.sync_copy(gathpf_v, outp_ref.at[m, pl.ds(wid * BPW, BPW)])


_sc_extract_cached = None


def _sc_extract(*args):
    global _sc_extract_cached
    if _sc_extract_cached is None:
        _sc_extract_cached = functools.partial(
            pl.kernel,
            out_type=(
                jax.ShapeDtypeStruct((2, BS * K), jnp.float32),
                jax.ShapeDtypeStruct((2, BS), jnp.float32),
            ),
            mesh=plsc.VectorSubcoreMesh(core_axis_name="c", subcore_axis_name="s"),
            scratch_types=[
                pltpu.VMEM((BPW, K), jnp.int32),
                pltpu.VMEM((BPW,), jnp.int32),
                pltpu.VMEM((NEG_PER_W,), jnp.int32),
                pltpu.VMEM((NEG_PER_W,), jnp.int32),
                pltpu.VMEM((BPW,), jnp.int32),
                pltpu.VMEM((BPW,), jnp.int32),
                pltpu.VMEM((NEG_PER_W,), jnp.int32),
                pltpu.VMEM((NEG_PER_W,), jnp.float32),
                pltpu.VMEM((BPW,), jnp.int32),
                pltpu.VMEM((BPW,), jnp.float32),
                pltpu.SemaphoreType.DMA,
            ],
        )(_sc_body)
    return _sc_extract_cached(*args)


def kernel(video_emb, audio_emb, y, view1_mem, view2_mem):
    y = y.astype(jnp.int32)

    bidx = jnp.arange(BS, dtype=jnp.int32)
    skey = y * BS + bidx
    rank = jnp.sum((skey[None, :] < skey[:, None]).astype(jnp.int32), axis=1)
    onehot = (rank[:, None] == bidx[None, :]).astype(jnp.int32)
    sy = jnp.sum(onehot * y[:, None], axis=0)
    perm = jnp.sum(onehot * bidx[:, None], axis=0)
    tiles = jnp.arange(NTILES + 1, dtype=jnp.int32)
    starts = jnp.sum(
        (y[None, :] < (tiles * ROWS_PER_TILE)[:, None]).astype(jnp.int32),
        axis=1)

    s1, s2, n1, n2 = pl.pallas_call(
        _tc_body,
        grid_spec=_TC_GRID,
        out_shape=[
            jax.ShapeDtypeStruct((NWORDS // 128, DIM), jnp.int32),
            jax.ShapeDtypeStruct((NWORDS // 128, DIM), jnp.int32),
            jax.ShapeDtypeStruct((MEM, DIM), jnp.float32),
            jax.ShapeDtypeStruct((MEM, DIM), jnp.float32),
        ],
    )(sy, perm, starts, video_emb, audio_emb, view1_mem, view2_mem)

    negs, pos = _sc_extract(
        s2.reshape(-1), s1.reshape(-1), y, _neg_sample_draw())

    scores = jnp.concatenate(
        [pos.reshape(2, BS, 1), negs.reshape(2, BS, K)], axis=2)
    return scores, n1, n2
